# baseline (speedup 1.0000x reference)
import functools

import jax
import jax.numpy as jnp
from jax import lax
from jax.experimental import pallas as pl
from jax.experimental.pallas import tpu as pltpu
from jax.experimental.pallas import tpu_sc as plsc

TABLE_ROWS = 100000
EMBED_D = 64
BATCH = 4096
SEQ = 26
NUM_CORES = 2
NUM_SUBCORES = 16
NW = NUM_CORES * NUM_SUBCORES
CHUNK = 128

_mesh = plsc.VectorSubcoreMesh(core_axis_name="c", subcore_axis_name="s")


@functools.partial(
    pl.kernel,
    mesh=_mesh,
    compiler_params=pltpu.CompilerParams(
        use_tc_tiling_on_sc=False,
        needs_layout_passes=False,
        disable_bounds_checks=True,
    ),
    out_type=jax.ShapeDtypeStruct((SEQ, 8, NW, 8, CHUNK), jnp.float32),
    scratch_types=[
        pltpu.VMEM((SEQ, CHUNK), jnp.int32),
        pltpu.VMEM((3, CHUNK, EMBED_D), jnp.float32),
        pltpu.VMEM((2, EMBED_D, 129), jnp.float32),
        pltpu.SemaphoreType.DMA,
        pltpu.SemaphoreType.DMA,
        pltpu.SemaphoreType.DMA,
        pltpu.SemaphoreType.DMA,
        pltpu.SemaphoreType.DMA,
    ],
)
def _gather_sc(
    idx_hbm, table_hbm, out_hbm, idx_v, rows_v, t_v, g0, g1, g2, w0, w1
):
    wid = lax.axis_index("s") * NUM_CORES + lax.axis_index("c")
    pltpu.sync_copy(idx_hbm.at[:, wid], idx_v)
    gs = (g0, g1, g2)
    ws = (w0, w1)
    lanes = lax.broadcasted_iota(jnp.int32, (16,), 0)
    dks = [d0 + lanes for d0 in (0, 16, 32, 48)]

    def transpose(rref, tref):
        def cstep(c0, carry):
            for ci in range(8):
                c = c0 * 8 + ci
                cvec = jnp.full((16,), c, jnp.int32)
                for k, d0 in enumerate((0, 16, 32, 48)):
                    x = rref[c, pl.ds(d0, 16)]
                    plsc.store_scatter(tref, [dks[k], cvec], x)
            return carry

        lax.fori_loop(0, 16, cstep, 0)

    def writeback(b, j, sem):
        for tr in range(8):
            pltpu.async_copy(
                t_v.at[b, pl.ds(tr * 8, 8), pl.ds(0, CHUNK)],
                out_hbm.at[j, tr, wid],
                sem,
            )

    def wb_wait(b, j, sem):
        for tr in range(8):
            pltpu.make_async_copy(
                t_v.at[b, pl.ds(tr * 8, 8), pl.ds(0, CHUNK)],
                out_hbm.at[j, tr, wid],
                sem,
            ).wait()

    for g in range(3):
        pltpu.async_copy(table_hbm.at[idx_v.at[g]], rows_v.at[g], gs[g])

    def outer(j0, carry):
        for u in range(6):
            j = 6 * j0 + u
            g = u % 3
            b = u % 2

            pltpu.make_async_copy(
                table_hbm.at[idx_v.at[j]], rows_v.at[g], gs[g]
            ).wait()

            @pl.when(j >= 2)
            def _reclaim():
                wb_wait(b, j - 2, ws[b])

            transpose(rows_v.at[g], t_v.at[b])

            writeback(b, j, ws[b])

            @pl.when(j < SEQ - 3)
            def _refill():
                pltpu.async_copy(
                    table_hbm.at[idx_v.at[j + 3]], rows_v.at[g], gs[g]
                )

        return carry

    lax.fori_loop(0, SEQ // 6, outer, 0)

    for j in range(SEQ - 2, SEQ):
        g = j % 3
        b = j % 2
        pltpu.make_async_copy(
            table_hbm.at[idx_v.at[j]], rows_v.at[g], gs[g]
        ).wait()
        wb_wait(b, j - 2, ws[b])
        transpose(rows_v.at[g], t_v.at[b])
        writeback(b, j, ws[b])

    for j in range(SEQ - 2, SEQ):
        wb_wait(j % 2, j, ws[j % 2])


def kernel(inputs, embeddings):
    idx = inputs.astype(jnp.int32).T.reshape(SEQ, NW, CHUNK)
    out = _gather_sc(idx, embeddings)
    return out.transpose(2, 4, 0, 1, 3).reshape(BATCH, SEQ, EMBED_D)


# speedup vs baseline: 1.0471x; 1.0471x over previous
import functools

import jax
import jax.numpy as jnp
from jax import lax
from jax.experimental import pallas as pl
from jax.experimental.pallas import tpu as pltpu
from jax.experimental.pallas import tpu_sc as plsc

TABLE_ROWS = 100000
EMBED_D = 64
BATCH = 4096
SEQ = 26
NUM_CORES = 2
NUM_SUBCORES = 16
NW = NUM_CORES * NUM_SUBCORES
CHUNK = 128

_mesh = plsc.VectorSubcoreMesh(core_axis_name="c", subcore_axis_name="s")


@functools.partial(
    pl.kernel,
    mesh=_mesh,
    compiler_params=pltpu.CompilerParams(
        use_tc_tiling_on_sc=False,
        needs_layout_passes=False,
        disable_bounds_checks=True,
    ),
    out_type=jax.ShapeDtypeStruct((SEQ, 8, NW, 8, CHUNK), jnp.float32),
    scratch_types=[
        pltpu.VMEM((SEQ, CHUNK), jnp.int32),
        pltpu.VMEM((2, CHUNK, EMBED_D), jnp.float32),
        pltpu.VMEM((2, 8, 8, 129), jnp.float32),
        pltpu.SemaphoreType.DMA,
        pltpu.SemaphoreType.DMA,
        pltpu.SemaphoreType.DMA,
        pltpu.SemaphoreType.DMA,
    ],
)
def _gather_sc(idx_hbm, table_hbm, out_hbm, idx_v, rows_v, t_v, g0, g1, w0, w1):
    wid = lax.axis_index("s") * NUM_CORES + lax.axis_index("c")
    pltpu.sync_copy(idx_hbm.at[:, wid], idx_v)
    gs = (g0, g1)
    ws = (w0, w1)
    lanes = lax.broadcasted_iota(jnp.int32, (16,), 0)
    trs = [(d0 + lanes) >> 3 for d0 in (0, 16, 32, 48)]
    rrs = [(d0 + lanes) & 7 for d0 in (0, 16, 32, 48)]

    def transpose(rref, tref):
        def cstep(c0, carry):
            for ci in range(8):
                c = c0 * 8 + ci
                cvec = jnp.full((16,), c, jnp.int32)
                for k in range(4):
                    x = rref[c, pl.ds(k * 16, 16)]
                    plsc.store_scatter(tref, [trs[k], rrs[k], cvec], x)
            return carry

        lax.fori_loop(0, 16, cstep, 0)

    def wb_copy(b, j):
        return pltpu.make_async_copy(
            t_v.at[b, :, :, pl.ds(0, CHUNK)], out_hbm.at[j, :, wid], ws[b]
        )

    for b in range(2):
        pltpu.async_copy(table_hbm.at[idx_v.at[b]], rows_v.at[b], gs[b])

    def outer(j0, carry):
        for b in range(2):
            j = 2 * j0 + b

            @pl.when(j0 > 0)
            def _reclaim():
                wb_copy(b, j - 2).wait()

            pltpu.make_async_copy(
                table_hbm.at[idx_v.at[j]], rows_v.at[b], gs[b]
            ).wait()

            transpose(rows_v.at[b], t_v.at[b])

            wb_copy(b, j).start()

            @pl.when(j < SEQ - 2)
            def _refill():
                pltpu.async_copy(
                    table_hbm.at[idx_v.at[j + 2]], rows_v.at[b], gs[b]
                )

        return carry

    lax.fori_loop(0, SEQ // 2, outer, 0)

    for b in range(2):
        wb_copy(b, SEQ - 2 + b).wait()


def kernel(inputs, embeddings):
    idx = inputs.astype(jnp.int32).T.reshape(SEQ, NW, CHUNK)
    out = _gather_sc(idx, embeddings)
    return out.transpose(2, 4, 0, 1, 3).reshape(BATCH, SEQ, EMBED_D)


# speedup vs baseline: 1.3396x; 1.2794x over previous
import functools

import jax
import jax.numpy as jnp
from jax import lax
from jax.experimental import pallas as pl
from jax.experimental.pallas import tpu as pltpu
from jax.experimental.pallas import tpu_sc as plsc

TABLE_ROWS = 100000
EMBED_D = 64
BATCH = 4096
SEQ = 26
NUM_CORES = 2
NUM_SUBCORES = 16
NW = NUM_CORES * NUM_SUBCORES
CHUNK = 128

_mesh = plsc.VectorSubcoreMesh(core_axis_name="c", subcore_axis_name="s")


@functools.partial(
    pl.kernel,
    mesh=_mesh,
    compiler_params=pltpu.CompilerParams(
        use_tc_tiling_on_sc=False,
        needs_layout_passes=False,
        disable_bounds_checks=True,
    ),
    out_type=jax.ShapeDtypeStruct((SEQ, 8, NW, 8, CHUNK), jnp.float32),
    scratch_types=[
        pltpu.VMEM((SEQ, CHUNK), jnp.int32),
        pltpu.VMEM((2, CHUNK, EMBED_D), jnp.float32),
        pltpu.VMEM((2, 8, 8, 129), jnp.float32),
        pltpu.SemaphoreType.DMA,
        pltpu.SemaphoreType.DMA,
        pltpu.SemaphoreType.DMA,
        pltpu.SemaphoreType.DMA,
    ],
)
def _gather_sc(idx_hbm, table_hbm, out_hbm, idx_v, rows_v, t_v, g0, g1, w0, w1):
    wid = lax.axis_index("s") * NUM_CORES + lax.axis_index("c")
    pltpu.sync_copy(idx_hbm.at[:, wid], idx_v)
    gs = (g0, g1)
    ws = (w0, w1)
    lanes = lax.broadcasted_iota(jnp.int32, (16,), 0)
    trs = [(d0 + lanes) >> 3 for d0 in (0, 16, 32, 48)]
    rrs = [(d0 + lanes) & 7 for d0 in (0, 16, 32, 48)]

    def transpose(rref, tref):
        @functools.partial(plsc.parallel_loop, 0, 16)
        def _cstep(c0):
            for ci in range(8):
                c = c0 * 8 + ci
                cvec = jnp.full((16,), c, jnp.int32)
                for k in range(4):
                    x = rref[c, pl.ds(k * 16, 16)]
                    plsc.store_scatter(tref, [trs[k], rrs[k], cvec], x)

    def wb_copy(b, j):
        return pltpu.make_async_copy(
            t_v.at[b, :, :, pl.ds(0, CHUNK)], out_hbm.at[j, :, wid], ws[b]
        )

    for b in range(2):
        pltpu.async_copy(table_hbm.at[idx_v.at[b]], rows_v.at[b], gs[b])

    def outer(j0, carry):
        for b in range(2):
            j = 2 * j0 + b

            @pl.when(j0 > 0)
            def _reclaim():
                wb_copy(b, j - 2).wait()

            pltpu.make_async_copy(
                table_hbm.at[idx_v.at[j]], rows_v.at[b], gs[b]
            ).wait()

            transpose(rows_v.at[b], t_v.at[b])

            wb_copy(b, j).start()

            @pl.when(j < SEQ - 2)
            def _refill():
                pltpu.async_copy(
                    table_hbm.at[idx_v.at[j + 2]], rows_v.at[b], gs[b]
                )

        return carry

    lax.fori_loop(0, SEQ // 2, outer, 0)

    for b in range(2):
        wb_copy(b, SEQ - 2 + b).wait()


def kernel(inputs, embeddings):
    idx = inputs.astype(jnp.int32).T.reshape(SEQ, NW, CHUNK)
    out = _gather_sc(idx, embeddings)
    return out.transpose(2, 4, 0, 1, 3).reshape(BATCH, SEQ, EMBED_D)
